# baseline (speedup 1.0000x reference)
import functools

import jax
import jax.numpy as jnp
from jax import lax
from jax.experimental import pallas as pl
from jax.experimental.pallas import tpu as pltpu
from jax.experimental.pallas import tpu_sc as plsc

LANES = 16


@functools.lru_cache(maxsize=None)
def _make_lookup(batch: int, nfeat: int, vocab: int):
    info = plsc.get_sparse_core_info()
    nw = info.num_cores * info.num_subcores
    cols = batch // nw
    npiece = 4
    piece = cols // npiece
    units = piece // LANES
    assert batch % nw == 0 and piece % LANES == 0

    mesh = plsc.VectorSubcoreMesh(core_axis_name="c", subcore_axis_name="s")

    @functools.partial(
        pl.kernel,
        mesh=mesh,
        out_type=jax.ShapeDtypeStruct((nfeat, batch), jnp.float32),
        scratch_types=[
            *[pltpu.VMEM((nfeat, piece), jnp.int32) for _ in range(npiece)],
            *[pltpu.VMEM((nfeat, piece), jnp.float32) for _ in range(npiece)],
            pltpu.VMEM((vocab, nfeat), jnp.float32),
            pltpu.SemaphoreType.DMA((npiece,)),
            pltpu.SemaphoreType.DMA((npiece,)),
        ],
        compiler_params=pltpu.CompilerParams(
            needs_layout_passes=False,
            disable_bounds_checks=True,
        ),
    )
    def lookup(tok_hbm, tbl_hbm, out_hbm, *refs):
        tok_bufs = refs[:npiece]
        out_bufs = refs[npiece : 2 * npiece]
        tbl_v = refs[2 * npiece]
        insem, outsem = refs[2 * npiece + 1], refs[2 * npiece + 2]
        wid = lax.axis_index("s") * info.num_cores + lax.axis_index("c")
        base = wid * cols
        in_copies = []
        for p in range(npiece):
            c = pltpu.make_async_copy(
                tok_hbm.at[:, pl.ds(base + p * piece, piece)],
                tok_bufs[p],
                insem.at[p],
            )
            c.start()
            in_copies.append(c)
        pltpu.sync_copy(tbl_hbm, tbl_v)
        out_copies = []
        for p in range(npiece):
            in_copies[p].wait()
            tok_v, out_v = tok_bufs[p], out_bufs[p]

            @plsc.parallel_loop(0, nfeat)
            def body(f, tok_v=tok_v, out_v=out_v):
                fvec = jnp.broadcast_to(f, (LANES,)).astype(jnp.int32)
                for c in range(units):
                    o = c * LANES
                    tok = tok_v[f, pl.ds(o, LANES)]
                    out_v[f, pl.ds(o, LANES)] = plsc.load_gather(
                        tbl_v, [tok, fvec]
                    )

            oc = pltpu.make_async_copy(
                out_v,
                out_hbm.at[:, pl.ds(base + p * piece, piece)],
                outsem.at[p],
            )
            oc.start()
            out_copies.append(oc)
        for oc in out_copies:
            oc.wait()

    return lookup


def kernel(inputs, mapping):
    tok = jnp.swapaxes(inputs.astype(jnp.int32), 0, 1)
    tbl = jnp.swapaxes(mapping.astype(jnp.float32), 0, 1)
    out = _make_lookup(inputs.shape[0], inputs.shape[1], mapping.shape[1])(
        tok, tbl
    )
    return jnp.swapaxes(out, 0, 1)


# speedup vs baseline: 1.0280x; 1.0280x over previous
import functools

import jax
import jax.numpy as jnp
from jax import lax
from jax.experimental import pallas as pl
from jax.experimental.pallas import tpu as pltpu
from jax.experimental.pallas import tpu_sc as plsc

LANES = 16


@functools.lru_cache(maxsize=None)
def _make_lookup(batch: int, nfeat: int, vocab: int):
    info = plsc.get_sparse_core_info()
    nw = info.num_cores * info.num_subcores
    cols = batch // nw
    units = cols // LANES
    assert batch % nw == 0 and cols % LANES == 0

    mesh = plsc.VectorSubcoreMesh(core_axis_name="c", subcore_axis_name="s")

    @functools.partial(
        pl.kernel,
        mesh=mesh,
        out_type=jax.ShapeDtypeStruct((nfeat, batch), jnp.float32),
        scratch_types=[
            pltpu.VMEM((nfeat, cols), jnp.int32),
            pltpu.VMEM((nfeat, cols), jnp.float32),
            pltpu.VMEM((vocab, nfeat), jnp.float32),
        ],
        compiler_params=pltpu.CompilerParams(
            needs_layout_passes=False,
            disable_bounds_checks=True,
        ),
    )
    def lookup(tok_hbm, tbl_hbm, out_hbm, tok_v, out_v, tbl_v):
        wid = lax.axis_index("s") * info.num_cores + lax.axis_index("c")
        base = wid * cols
        with jax.named_scope("dma_in"):
            pltpu.sync_copy(tok_hbm.at[:, pl.ds(base, cols)], tok_v)
        with jax.named_scope("dma_tbl"):
            pltpu.sync_copy(tbl_hbm, tbl_v)

        with jax.named_scope("compute"):

            @plsc.parallel_loop(0, nfeat)
            def body(f):
                fvec = jnp.broadcast_to(f, (LANES,)).astype(jnp.int32)
                for c in range(units):
                    o = c * LANES
                    tok = tok_v[f, pl.ds(o, LANES)]
                    vals = plsc.load_gather(tbl_v, [tok, fvec])
                    out_v[f, pl.ds(o, LANES)] = vals

        with jax.named_scope("dma_out"):
            pltpu.sync_copy(out_v, out_hbm.at[:, pl.ds(base, cols)])

    return lookup


def kernel(inputs, mapping):
    tok = jnp.swapaxes(inputs.astype(jnp.int32), 0, 1)
    tbl = jnp.swapaxes(mapping.astype(jnp.float32), 0, 1)
    out = _make_lookup(inputs.shape[0], inputs.shape[1], mapping.shape[1])(
        tok, tbl
    )
    return jnp.swapaxes(out, 0, 1)


# speedup vs baseline: 1.2359x; 1.2021x over previous
import functools

import jax
import jax.numpy as jnp
from jax import lax
from jax.experimental import pallas as pl
from jax.experimental.pallas import tpu as pltpu
from jax.experimental.pallas import tpu_sc as plsc

LANES = 16


@functools.lru_cache(maxsize=None)
def _make_lookup(batch: int, nfeat: int, vocab: int):
    info = plsc.get_sparse_core_info()
    nw = info.num_cores * info.num_subcores
    cols = batch // nw
    units = cols // LANES
    assert batch % nw == 0 and cols % LANES == 0 and vocab == LANES

    mesh = plsc.VectorSubcoreMesh(core_axis_name="c", subcore_axis_name="s")

    @functools.partial(
        pl.kernel,
        mesh=mesh,
        out_type=jax.ShapeDtypeStruct((nfeat, batch), jnp.float32),
        scratch_types=[
            pltpu.VMEM((nfeat, cols), jnp.int32),
            pltpu.VMEM((nfeat, cols), jnp.float32),
            pltpu.VMEM((vocab, nfeat), jnp.float32),
            pltpu.VMEM((nfeat, vocab), jnp.float32),
            pltpu.SemaphoreType.DMA,
        ],
        compiler_params=pltpu.CompilerParams(
            needs_layout_passes=False,
            disable_bounds_checks=True,
        ),
    )
    def lookup(tok_hbm, tbl_hbm, out_hbm, tok_v, out_v, tblT_v, tbl_v, sem):
        wid = lax.axis_index("s") * info.num_cores + lax.axis_index("c")
        base = wid * cols
        with jax.named_scope("dma_in"):
            inc = pltpu.make_async_copy(
                tok_hbm.at[:, pl.ds(base, cols)], tok_v, sem
            )
            inc.start()
        with jax.named_scope("dma_tbl"):
            pltpu.sync_copy(tbl_hbm, tblT_v)
            vocab_iota = lax.iota(jnp.int32, LANES)
            for f in range(nfeat):
                fvec = jnp.broadcast_to(jnp.int32(f), (LANES,))
                col = plsc.load_gather(tblT_v, [vocab_iota, fvec])
                tbl_v[f, pl.ds(0, LANES)] = col
            inc.wait()

        with jax.named_scope("compute"):

            @plsc.parallel_loop(0, nfeat)
            def body(f):
                tbl_row = tbl_v.at[f]
                for c in range(units):
                    o = c * LANES
                    tok = tok_v[f, pl.ds(o, LANES)]
                    vals = plsc.load_gather(tbl_row, [tok])
                    out_v[f, pl.ds(o, LANES)] = vals

        with jax.named_scope("dma_out"):
            pltpu.sync_copy(out_v, out_hbm.at[:, pl.ds(base, cols)])

    return lookup


def kernel(inputs, mapping):
    tok = jnp.swapaxes(inputs.astype(jnp.int32), 0, 1)
    tbl = jnp.swapaxes(mapping.astype(jnp.float32), 0, 1)
    out = _make_lookup(inputs.shape[0], inputs.shape[1], mapping.shape[1])(
        tok, tbl
    )
    return jnp.swapaxes(out, 0, 1)


# speedup vs baseline: 1.2389x; 1.0025x over previous
import functools

import jax
import jax.numpy as jnp
from jax import lax
from jax.experimental import pallas as pl
from jax.experimental.pallas import tpu as pltpu
from jax.experimental.pallas import tpu_sc as plsc

LANES = 16


@functools.lru_cache(maxsize=None)
def _make_lookup(batch: int, nfeat: int, vocab: int):
    info = plsc.get_sparse_core_info()
    nw = info.num_cores * info.num_subcores
    cols = batch // nw
    units = cols // LANES
    assert batch % nw == 0 and cols % LANES == 0 and vocab == LANES

    mesh = plsc.VectorSubcoreMesh(core_axis_name="c", subcore_axis_name="s")

    @functools.partial(
        pl.kernel,
        mesh=mesh,
        out_type=jax.ShapeDtypeStruct((nfeat, batch), jnp.float32),
        scratch_types=[
            pltpu.VMEM((nfeat, cols), jnp.int32),
            pltpu.VMEM((nfeat, cols), jnp.float32),
            pltpu.VMEM((vocab, nfeat), jnp.float32),
            pltpu.VMEM((nfeat, vocab), jnp.float32),
            pltpu.SemaphoreType.DMA,
            pltpu.SemaphoreType.DMA,
        ],
        compiler_params=pltpu.CompilerParams(
            needs_layout_passes=False,
            disable_bounds_checks=True,
        ),
    )
    def lookup(
        tok_hbm, tbl_hbm, out_hbm, tok_v, out_v, tblT_v, tbl_v, tsem, sem
    ):
        wid = lax.axis_index("s") * info.num_cores + lax.axis_index("c")
        base = wid * cols
        with jax.named_scope("dma_start"):
            tc = pltpu.make_async_copy(tbl_hbm, tblT_v, tsem)
            tc.start()
            inc = pltpu.make_async_copy(
                tok_hbm.at[:, pl.ds(base, cols)], tok_v, sem
            )
            inc.start()
        with jax.named_scope("dma_tbl"):
            tc.wait()
            vocab_iota = lax.iota(jnp.int32, LANES)
            for f in range(nfeat):
                fvec = jnp.broadcast_to(jnp.int32(f), (LANES,))
                col = plsc.load_gather(tblT_v, [vocab_iota, fvec])
                tbl_v[f, pl.ds(0, LANES)] = col
        with jax.named_scope("dma_in"):
            inc.wait()

        with jax.named_scope("compute"):

            @plsc.parallel_loop(0, nfeat)
            def body(f):
                tbl_row = tbl_v.at[f]
                for c in range(units):
                    o = c * LANES
                    tok = tok_v[f, pl.ds(o, LANES)]
                    vals = plsc.load_gather(tbl_row, [tok])
                    out_v[f, pl.ds(o, LANES)] = vals

        with jax.named_scope("dma_out"):
            pltpu.sync_copy(out_v, out_hbm.at[:, pl.ds(base, cols)])

    return lookup


def kernel(inputs, mapping):
    tok = jnp.swapaxes(inputs.astype(jnp.int32), 0, 1)
    tbl = jnp.swapaxes(mapping.astype(jnp.float32), 0, 1)
    out = _make_lookup(inputs.shape[0], inputs.shape[1], mapping.shape[1])(
        tok, tbl
    )
    return jnp.swapaxes(out, 0, 1)
